# baseline (device time: 16931 ns/iter reference)
import jax
import jax.numpy as jnp
from jax import lax
from jax.experimental import pallas as pl
from jax.experimental.pallas import tpu as pltpu

EPS = 1e-5
N_GLOBAL = 2048
NCHUNK = 8
LANES = 128
NHALF = NCHUNK // 2


def kernel(x, gamma):
    m, n_local = x.shape
    rows = m // NCHUNK
    srows = m // LANES
    spc = srows // NCHUNK
    shalf = srows // 2
    g2d = gamma.reshape(1, n_local)

    def body(x_hbm, g_ref, out_hbm, x_vmem, out_vmem, comm_ref,
             in_sems, out_sems, send_sems, recv_sems):
        my_x = lax.axis_index("x")
        my_y = lax.axis_index("y")
        nbr = (my_x, 1 - my_y)

        barrier_sem = pltpu.get_barrier_semaphore()
        pl.semaphore_signal(
            barrier_sem, inc=1, device_id=nbr,
            device_id_type=pl.DeviceIdType.MESH,
        )

        def half_rdma(h):
            sl = slice(h * shalf, (h + 1) * shalf)
            return pltpu.make_async_remote_copy(
                src_ref=comm_ref.at[0, sl, :],
                dst_ref=comm_ref.at[1, sl, :],
                send_sem=send_sems.at[h],
                recv_sem=recv_sems.at[h],
                device_id=nbr,
                device_id_type=pl.DeviceIdType.MESH,
            )

        in_copies = []
        for i in range(NCHUNK):
            sl = slice(i * rows, (i + 1) * rows)
            cp = pltpu.make_async_copy(
                x_hbm.at[sl, :], x_vmem.at[sl, :], in_sems.at[i],
            )
            cp.start()
            in_copies.append(cp)

        ones_k = jnp.ones((n_local, LANES), jnp.float32)
        rdmas = []
        for i in range(NCHUNK):
            sl = slice(i * rows, (i + 1) * rows)
            in_copies[i].wait()
            xc = x_vmem[sl, :]
            xsq = xc * xc
            psum = lax.dot_general(
                xsq, ones_k, (((1,), (0,)), ((), ())),
                preferred_element_type=jnp.float32,
            )
            comm_ref[0, slice(i * spc, (i + 1) * spc), :] = (
                psum[:, 0:1].reshape(spc, LANES)
            )
            if i == NHALF - 1:
                pl.semaphore_wait(barrier_sem, 1)
                r = half_rdma(0)
                r.start()
                rdmas.append(r)
            elif i == NCHUNK - 1:
                r = half_rdma(1)
                r.start()
                rdmas.append(r)

        g = g_ref[:, :]

        out_copies = []
        for h in range(2):
            rdmas[h].wait()
            sl = slice(h * shalf, (h + 1) * shalf)
            total = comm_ref[0, sl, :] + comm_ref[1, sl, :]
            ct = lax.rsqrt(total / N_GLOBAL + EPS).T
            for j in range(NHALF):
                i = h * NHALF + j
                rsl = slice(i * rows, (i + 1) * rows)
                for k in range(spc):
                    gsl = slice(i * rows + k * LANES,
                                i * rows + (k + 1) * LANES)
                    ck = j * spc + k
                    out_vmem[gsl, :] = (
                        g * x_vmem[gsl, :] * ct[:, ck:ck + 1]
                    )
                cp = pltpu.make_async_copy(
                    out_vmem.at[rsl, :], out_hbm.at[rsl, :], out_sems.at[i],
                )
                cp.start()
                out_copies.append(cp)
        for cp in out_copies:
            cp.wait()

    return pl.pallas_call(
        body,
        out_shape=jax.ShapeDtypeStruct((m, n_local), x.dtype),
        in_specs=[
            pl.BlockSpec(memory_space=pl.ANY),
            pl.BlockSpec(memory_space=pltpu.VMEM),
        ],
        out_specs=pl.BlockSpec(memory_space=pl.ANY),
        scratch_shapes=[
            pltpu.VMEM((m, n_local), jnp.float32),
            pltpu.VMEM((m, n_local), jnp.float32),
            pltpu.VMEM((2, m // LANES, LANES), jnp.float32),
            pltpu.SemaphoreType.DMA((NCHUNK,)),
            pltpu.SemaphoreType.DMA((NCHUNK,)),
            pltpu.SemaphoreType.DMA((2,)),
            pltpu.SemaphoreType.DMA((2,)),
        ],
        compiler_params=pltpu.CompilerParams(collective_id=0),
    )(x, g2d)
